# baseline (speedup 1.0000x reference)
import functools

import jax
import jax.numpy as jnp
from jax import lax
from jax.experimental import pallas as pl
from jax.experimental.pallas import tpu as pltpu
from jax.experimental.pallas import tpu_sc as plsc

N = 10000
E = 320000
D = 128
NUM_LAYERS = 2
BN_EPS = 1e-5

NPAD = 10240
ROWB = 1024
NBLK = NPAD // ROWB
NCORES = 2
NSUB = 16
NW = NCORES * NSUB
CHUNK = 64
NBUF = 4
G = 32
NG = 5
CPT = NG * G
EPT = CPT * CHUNK
EPAD = NW * EPT
RPT = NPAD // NSUB
DUMMY = N


def _idx_body(src_ref, attr_ref, idx_ref):
    idx_ref[...] = src_ref[...] * 4 + attr_ref[...]


def _make_idx(src_r, attr_r):
    return pl.pallas_call(
        _idx_body,
        out_shape=jax.ShapeDtypeStruct((NSUB, 2 * NG, G, CHUNK), jnp.int32),
    )(src_r, attr_r)


def _mtab0_body(x_ref, bond_ref, out_ref):
    h = x_ref[...]
    for a in range(4):
        out_ref[:, a * D:(a + 1) * D] = jnp.maximum(h + bond_ref[a:a + 1, :], 0.0)


def _make_mtab0(x_pad, bond):
    return pl.pallas_call(
        _mtab0_body,
        grid=(NBLK,),
        in_specs=[
            pl.BlockSpec((ROWB, D), lambda i: (i, 0)),
            pl.BlockSpec((8, D), lambda i: (0, 0)),
        ],
        out_shape=jax.ShapeDtypeStruct((NPAD, 4 * D), jnp.float32),
        out_specs=pl.BlockSpec((ROWB, 4 * D), lambda i: (i, 0)),
    )(x_pad, bond)


def _mlp_core(x_ref, p0_ref, p1_ref, w1_ref, w2_ref, c_ref):
    z = x_ref[...] * c_ref[6:7, :] + p0_ref[0] + p1_ref[0]
    z = jnp.dot(z, w1_ref[...], preferred_element_type=jnp.float32) + c_ref[0:1, :]
    z = jnp.maximum(z * c_ref[2:3, :] + c_ref[3:4, :], 0.0)
    z = jnp.dot(z, w2_ref[...], preferred_element_type=jnp.float32) + c_ref[1:2, :]
    return z * c_ref[4:5, :] + c_ref[5:6, :]


def _mlp_mid_body(x_ref, p0_ref, p1_ref, w1_ref, w2_ref, c_ref, bond_ref,
                  h_ref, mt_ref):
    z = jnp.maximum(_mlp_core(x_ref, p0_ref, p1_ref, w1_ref, w2_ref, c_ref), 0.0)
    h_ref[...] = z
    for a in range(4):
        mt_ref[:, a * D:(a + 1) * D] = jnp.maximum(z + bond_ref[a:a + 1, :], 0.0)


def _mlp_last_body(x_ref, p0_ref, p1_ref, w1_ref, w2_ref, c_ref, h_ref):
    h_ref[...] = _mlp_core(x_ref, p0_ref, p1_ref, w1_ref, w2_ref, c_ref)


def _mlp_mid(h, partials, w1, w2, coefs, bond_next):
    return pl.pallas_call(
        _mlp_mid_body,
        grid=(NBLK,),
        in_specs=[
            pl.BlockSpec((ROWB, D), lambda i: (i, 0)),
            pl.BlockSpec((1, ROWB, D), lambda i: (0, i, 0)),
            pl.BlockSpec((1, ROWB, D), lambda i: (1, i, 0)),
            pl.BlockSpec((D, D), lambda i: (0, 0)),
            pl.BlockSpec((D, D), lambda i: (0, 0)),
            pl.BlockSpec((8, D), lambda i: (0, 0)),
            pl.BlockSpec((8, D), lambda i: (0, 0)),
        ],
        out_shape=[
            jax.ShapeDtypeStruct((NPAD, D), jnp.float32),
            jax.ShapeDtypeStruct((NPAD, 4 * D), jnp.float32),
        ],
        out_specs=[
            pl.BlockSpec((ROWB, D), lambda i: (i, 0)),
            pl.BlockSpec((ROWB, 4 * D), lambda i: (i, 0)),
        ],
    )(h, partials, partials, w1, w2, coefs, bond_next)


def _mlp_last(h, partials, w1, w2, coefs):
    return pl.pallas_call(
        _mlp_last_body,
        grid=(NBLK,),
        in_specs=[
            pl.BlockSpec((ROWB, D), lambda i: (i, 0)),
            pl.BlockSpec((1, ROWB, D), lambda i: (0, i, 0)),
            pl.BlockSpec((1, ROWB, D), lambda i: (1, i, 0)),
            pl.BlockSpec((D, D), lambda i: (0, 0)),
            pl.BlockSpec((D, D), lambda i: (0, 0)),
            pl.BlockSpec((8, D), lambda i: (0, 0)),
        ],
        out_shape=jax.ShapeDtypeStruct((NPAD, D), jnp.float32),
        out_specs=pl.BlockSpec((ROWB, D), lambda i: (i, 0)),
    )(h, partials, partials, w1, w2, coefs)


def _sc_scatter(mtab, idx_r, dst_r, zeros):
    mesh = plsc.VectorSubcoreMesh(core_axis_name="c", subcore_axis_name="s")

    @functools.partial(
        pl.kernel,
        out_type=jax.ShapeDtypeStruct((NCORES, NPAD, D), jnp.float32),
        mesh=mesh,
        scratch_types=[
            pltpu.VMEM((G, CHUNK), jnp.int32),
            pltpu.VMEM((G, CHUNK), jnp.int32),
            [pltpu.VMEM((CHUNK, D), jnp.float32) for _ in range(NBUF)],
            pltpu.VMEM_SHARED((NPAD, D), jnp.float32),
            [pltpu.SemaphoreType.DMA for _ in range(NBUF)],
        ],
    )
    def body(mtab_hbm, idx_hbm, dst_hbm, zeros_hbm, out_hbm,
             idx_v, dst_v, bufs, agg_sh, sems):
        cid = lax.axis_index("c")
        sid = lax.axis_index("s")
        ACT = 0
        wid = cid * NSUB + sid
        pltpu.sync_copy(zeros_hbm.at[pl.ds(sid * RPT, RPT)],
                        agg_sh.at[pl.ds(sid * RPT, RPT)])
        plsc.subcore_barrier()

        def group(g, carry):
            pltpu.sync_copy(idx_hbm.at[sid, g], idx_v)
            pltpu.sync_copy(dst_hbm.at[sid, g], dst_v)
            for c in range(NBUF - 1):
                pltpu.async_copy(mtab_hbm.at[idx_v.at[c]], bufs[c], sems[c])

            def step(t, inner):
                for b in range(NBUF):
                    c = NBUF * t + b
                    nc = c + NBUF - 1

                    nb = (b + NBUF - 1) % NBUF

                    @pl.when(nc < G)
                    def _():
                        pltpu.async_copy(mtab_hbm.at[idx_v.at[nc]],
                                         bufs[nb], sems[nb])

                    pltpu.make_async_copy(mtab_hbm.at[idx_v.at[c]],
                                          bufs[b], sems[b]).wait()
                    pltpu.sync_copy(bufs[b], agg_sh.at[dst_v.at[c]], add=True)
                return inner

            lax.fori_loop(0, G // NBUF, step, 0)
            return carry

        @pl.when(cid == ACT)
        def _():
            lax.fori_loop(0, 2 * NG, group, 0)

        plsc.subcore_barrier()
        pltpu.sync_copy(agg_sh.at[pl.ds(sid * RPT, RPT)],
                        out_hbm.at[cid, pl.ds(sid * RPT, RPT)])

    return body(mtab, idx_r, dst_r, zeros)


def kernel(x, params, edge_index, edge_attr):
    src = edge_index[0].astype(jnp.int32)
    dst = edge_index[1].astype(jnp.int32)
    attr = edge_attr.astype(jnp.int32)

    pad = EPAD - E
    src_r = jnp.pad(src, (0, pad)).reshape(NSUB, 2 * NG, G, CHUNK)
    attr_r = jnp.pad(attr, (0, pad)).reshape(NSUB, 2 * NG, G, CHUNK)
    pad_dst = DUMMY + (jnp.arange(pad, dtype=jnp.int32) % (NPAD - N))
    dst_r = jnp.concatenate([dst, pad_dst]).reshape(NSUB, 2 * NG, G, CHUNK)

    idx_r = _make_idx(src_r, attr_r)

    h = jnp.pad(x, ((0, NPAD - N), (0, 0)))
    zeros = jnp.zeros((NPAD, D), jnp.float32)
    bn_s = jnp.float32(1.0 / jnp.sqrt(1.0 + BN_EPS))

    mtab = None
    for l in range(NUM_LAYERS):
        coefs = jnp.stack([
            params[f"b1_{l}"],
            params[f"b2_{l}"],
            params[f"mg_{l}"] * bn_s,
            params[f"mb_{l}"],
            params[f"og_{l}"] * bn_s,
            params[f"ob_{l}"],
            jnp.full((D,), 1.0, jnp.float32) * (1.0 + params[f"eps_{l}"][0]),
            jnp.zeros((D,), jnp.float32),
        ])
        if l == 0:
            bond0 = jnp.pad(params["bond_0"], ((0, 4), (0, 0)))
            mtab = _make_mtab0(h, bond0)
        partials = _sc_scatter(mtab.reshape(NPAD * 4, D), idx_r, dst_r, zeros)
        if l == NUM_LAYERS - 1:
            h = _mlp_last(h, partials, params[f"W1_{l}"], params[f"W2_{l}"], coefs)
        else:
            bond_next = jnp.pad(params[f"bond_{l + 1}"], ((0, 4), (0, 0)))
            h, mtab = _mlp_mid(h, partials, params[f"W1_{l}"], params[f"W2_{l}"],
                               coefs, bond_next)
    return h[:N]


# speedup vs baseline: 3.8194x; 3.8194x over previous
import functools

import jax
import jax.numpy as jnp
from jax import lax
from jax.experimental import pallas as pl
from jax.experimental.pallas import tpu as pltpu
from jax.experimental.pallas import tpu_sc as plsc

N = 10000
E = 320000
D = 128
NUM_LAYERS = 2
BN_EPS = 1e-5

NPAD = 10240
ROWB = 1024
NBLK = NPAD // ROWB
NCORES = 2
NSUB = 16
NW = NCORES * NSUB
CHUNK = 64
NBUF = 4
G = 32
NG = 5
CPT = NG * G
EPT = CPT * CHUNK
EPAD = NW * EPT
RPT = NPAD // NSUB
DUMMY = N


def _idx_body(src_ref, attr_ref, idx_ref):
    idx_ref[...] = src_ref[...] * 4 + attr_ref[...]


def _make_idx(src_r, attr_r):
    return pl.pallas_call(
        _idx_body,
        out_shape=jax.ShapeDtypeStruct((NW, NG, G, CHUNK), jnp.int32),
    )(src_r, attr_r)


def _mtab0_body(x_ref, bond_ref, out_ref):
    h = x_ref[...]
    for a in range(4):
        out_ref[:, a * D:(a + 1) * D] = jnp.maximum(h + bond_ref[a:a + 1, :], 0.0)


def _make_mtab0(x_pad, bond):
    return pl.pallas_call(
        _mtab0_body,
        grid=(NBLK,),
        in_specs=[
            pl.BlockSpec((ROWB, D), lambda i: (i, 0)),
            pl.BlockSpec((8, D), lambda i: (0, 0)),
        ],
        out_shape=jax.ShapeDtypeStruct((NPAD, 4 * D), jnp.float32),
        out_specs=pl.BlockSpec((ROWB, 4 * D), lambda i: (i, 0)),
    )(x_pad, bond)


def _mlp_core(x_ref, p0_ref, p1_ref, w1_ref, w2_ref, c_ref):
    z = x_ref[...] * c_ref[6:7, :] + p0_ref[0] + p1_ref[0]
    z = jnp.dot(z, w1_ref[...], preferred_element_type=jnp.float32) + c_ref[0:1, :]
    z = jnp.maximum(z * c_ref[2:3, :] + c_ref[3:4, :], 0.0)
    z = jnp.dot(z, w2_ref[...], preferred_element_type=jnp.float32) + c_ref[1:2, :]
    return z * c_ref[4:5, :] + c_ref[5:6, :]


def _mlp_mid_body(x_ref, p0_ref, p1_ref, w1_ref, w2_ref, c_ref, bond_ref,
                  h_ref, mt_ref):
    z = jnp.maximum(_mlp_core(x_ref, p0_ref, p1_ref, w1_ref, w2_ref, c_ref), 0.0)
    h_ref[...] = z
    for a in range(4):
        mt_ref[:, a * D:(a + 1) * D] = jnp.maximum(z + bond_ref[a:a + 1, :], 0.0)


def _mlp_last_body(x_ref, p0_ref, p1_ref, w1_ref, w2_ref, c_ref, h_ref):
    h_ref[...] = _mlp_core(x_ref, p0_ref, p1_ref, w1_ref, w2_ref, c_ref)


def _mlp_mid(h, partials, w1, w2, coefs, bond_next):
    return pl.pallas_call(
        _mlp_mid_body,
        grid=(NBLK,),
        in_specs=[
            pl.BlockSpec((ROWB, D), lambda i: (i, 0)),
            pl.BlockSpec((1, ROWB, D), lambda i: (0, i, 0)),
            pl.BlockSpec((1, ROWB, D), lambda i: (1, i, 0)),
            pl.BlockSpec((D, D), lambda i: (0, 0)),
            pl.BlockSpec((D, D), lambda i: (0, 0)),
            pl.BlockSpec((8, D), lambda i: (0, 0)),
            pl.BlockSpec((8, D), lambda i: (0, 0)),
        ],
        out_shape=[
            jax.ShapeDtypeStruct((NPAD, D), jnp.float32),
            jax.ShapeDtypeStruct((NPAD, 4 * D), jnp.float32),
        ],
        out_specs=[
            pl.BlockSpec((ROWB, D), lambda i: (i, 0)),
            pl.BlockSpec((ROWB, 4 * D), lambda i: (i, 0)),
        ],
    )(h, partials, partials, w1, w2, coefs, bond_next)


def _mlp_last(h, partials, w1, w2, coefs):
    return pl.pallas_call(
        _mlp_last_body,
        grid=(NBLK,),
        in_specs=[
            pl.BlockSpec((ROWB, D), lambda i: (i, 0)),
            pl.BlockSpec((1, ROWB, D), lambda i: (0, i, 0)),
            pl.BlockSpec((1, ROWB, D), lambda i: (1, i, 0)),
            pl.BlockSpec((D, D), lambda i: (0, 0)),
            pl.BlockSpec((D, D), lambda i: (0, 0)),
            pl.BlockSpec((8, D), lambda i: (0, 0)),
        ],
        out_shape=jax.ShapeDtypeStruct((NPAD, D), jnp.float32),
        out_specs=pl.BlockSpec((ROWB, D), lambda i: (i, 0)),
    )(h, partials, partials, w1, w2, coefs)


def _sc_scatter(mtab, idx_r, dst_r, zeros):
    mesh = plsc.VectorSubcoreMesh(core_axis_name="c", subcore_axis_name="s")

    @functools.partial(
        pl.kernel,
        out_type=jax.ShapeDtypeStruct((NCORES, NPAD, D), jnp.float32),
        mesh=mesh,
        scratch_types=[
            pltpu.VMEM((G, CHUNK), jnp.int32),
            pltpu.VMEM((G, CHUNK), jnp.int32),
            [pltpu.VMEM((CHUNK, D), jnp.float32) for _ in range(NBUF)],
            pltpu.VMEM_SHARED((NPAD, D), jnp.float32),
            [pltpu.SemaphoreType.DMA for _ in range(NBUF)],
        ],
    )
    def body(mtab_hbm, idx_hbm, dst_hbm, zeros_hbm, out_hbm,
             idx_v, dst_v, bufs, agg_sh, sems):
        cid = lax.axis_index("c")
        sid = lax.axis_index("s")
        wid = cid * NSUB + sid
        pltpu.sync_copy(zeros_hbm.at[pl.ds(sid * RPT, RPT)],
                        agg_sh.at[pl.ds(sid * RPT, RPT)])
        plsc.subcore_barrier()

        def group(g, carry):
            pltpu.sync_copy(idx_hbm.at[wid, g], idx_v)
            pltpu.sync_copy(dst_hbm.at[wid, g], dst_v)
            for c in range(NBUF - 1):
                pltpu.async_copy(mtab_hbm.at[idx_v.at[c]], bufs[c], sems[c])

            def step(t, inner):
                for b in range(NBUF):
                    c = NBUF * t + b
                    nc = c + NBUF - 1

                    nb = (b + NBUF - 1) % NBUF

                    @pl.when(nc < G)
                    def _():
                        pltpu.async_copy(mtab_hbm.at[idx_v.at[nc]],
                                         bufs[nb], sems[nb])

                    pltpu.make_async_copy(mtab_hbm.at[idx_v.at[c]],
                                          bufs[b], sems[b]).wait()
                    pltpu.sync_copy(bufs[b], agg_sh.at[dst_v.at[c]], add=True)
                return inner

            lax.fori_loop(0, G // NBUF, step, 0)
            return carry

        lax.fori_loop(0, NG, group, 0)
        plsc.subcore_barrier()
        pltpu.sync_copy(agg_sh.at[pl.ds(sid * RPT, RPT)],
                        out_hbm.at[cid, pl.ds(sid * RPT, RPT)])

    return body(mtab, idx_r, dst_r, zeros)


def kernel(x, params, edge_index, edge_attr):
    src = edge_index[0].astype(jnp.int32)
    dst = edge_index[1].astype(jnp.int32)
    attr = edge_attr.astype(jnp.int32)

    pad = EPAD - E
    pad_iota = jnp.arange(pad, dtype=jnp.int32)
    src_r = jnp.concatenate([src, pad_iota % N]).reshape(NW, NG, G, CHUNK)
    attr_r = jnp.pad(attr, (0, pad)).reshape(NW, NG, G, CHUNK)
    pad_dst = DUMMY + (pad_iota % (NPAD - N))
    dst_r = jnp.concatenate([dst, pad_dst]).reshape(NW, NG, G, CHUNK)

    idx_r = _make_idx(src_r, attr_r)

    h = jnp.pad(x, ((0, NPAD - N), (0, 0)))
    zeros = jnp.zeros((NPAD, D), jnp.float32)
    bn_s = jnp.float32(1.0 / jnp.sqrt(1.0 + BN_EPS))

    mtab = None
    for l in range(NUM_LAYERS):
        coefs = jnp.stack([
            params[f"b1_{l}"],
            params[f"b2_{l}"],
            params[f"mg_{l}"] * bn_s,
            params[f"mb_{l}"],
            params[f"og_{l}"] * bn_s,
            params[f"ob_{l}"],
            jnp.full((D,), 1.0, jnp.float32) * (1.0 + params[f"eps_{l}"][0]),
            jnp.zeros((D,), jnp.float32),
        ])
        if l == 0:
            bond0 = jnp.pad(params["bond_0"], ((0, 4), (0, 0)))
            mtab = _make_mtab0(h, bond0)
        partials = _sc_scatter(mtab.reshape(NPAD * 4, D), idx_r, dst_r, zeros)
        if l == NUM_LAYERS - 1:
            h = _mlp_last(h, partials, params[f"W1_{l}"], params[f"W2_{l}"], coefs)
        else:
            bond_next = jnp.pad(params[f"bond_{l + 1}"], ((0, 4), (0, 0)))
            h, mtab = _mlp_mid(h, partials, params[f"W1_{l}"], params[f"W2_{l}"],
                               coefs, bond_next)
    return h[:N]


# speedup vs baseline: 4.4380x; 1.1620x over previous
import functools

import jax
import jax.numpy as jnp
from jax import lax
from jax.experimental import pallas as pl
from jax.experimental.pallas import tpu as pltpu
from jax.experimental.pallas import tpu_sc as plsc

N = 10000
E = 320000
D = 128
NUM_LAYERS = 2
BN_EPS = 1e-5

NPAD = 10240
ROWB = 1024
NBLK = NPAD // ROWB
NCORES = 2
NSUB = 16
NW = NCORES * NSUB
CHUNK = 64
NBUF = 4
G = 32
NG = 5
CPT = NG * G
EPT = CPT * CHUNK
EPAD = NW * EPT
RPT = NPAD // NSUB
DUMMY = N


def _idx_body(src_ref, attr_ref, idx_ref):
    idx_ref[...] = attr_ref[...] * NPAD + src_ref[...]


def _make_idx(src_r, attr_r):
    return pl.pallas_call(
        _idx_body,
        out_shape=jax.ShapeDtypeStruct((NW, NG, G, CHUNK), jnp.int32),
    )(src_r, attr_r)


def _mtab0_body(x_ref, bond_ref, out_ref):
    h = x_ref[...]
    for a in range(4):
        out_ref[a] = jnp.maximum(h + bond_ref[a:a + 1, :], 0.0)


def _make_mtab0(x_pad, bond):
    return pl.pallas_call(
        _mtab0_body,
        grid=(NBLK,),
        in_specs=[
            pl.BlockSpec((ROWB, D), lambda i: (i, 0)),
            pl.BlockSpec((8, D), lambda i: (0, 0)),
        ],
        out_shape=jax.ShapeDtypeStruct((4, NPAD, D), jnp.float32),
        out_specs=pl.BlockSpec((4, ROWB, D), lambda i: (0, i, 0)),
    )(x_pad, bond)


def _mlp_core(x_ref, p0_ref, p1_ref, w1_ref, w2_ref, c_ref):
    z = x_ref[...] * c_ref[6:7, :] + p0_ref[0] + p1_ref[0]
    z = jnp.dot(z, w1_ref[...], preferred_element_type=jnp.float32) + c_ref[0:1, :]
    z = jnp.maximum(z * c_ref[2:3, :] + c_ref[3:4, :], 0.0)
    z = jnp.dot(z, w2_ref[...], preferred_element_type=jnp.float32) + c_ref[1:2, :]
    return z * c_ref[4:5, :] + c_ref[5:6, :]


def _mlp_mid_body(x_ref, p0_ref, p1_ref, w1_ref, w2_ref, c_ref, bond_ref,
                  h_ref, mt_ref):
    z = jnp.maximum(_mlp_core(x_ref, p0_ref, p1_ref, w1_ref, w2_ref, c_ref), 0.0)
    h_ref[...] = z
    for a in range(4):
        mt_ref[a] = jnp.maximum(z + bond_ref[a:a + 1, :], 0.0)


def _mlp_last_body(x_ref, p0_ref, p1_ref, w1_ref, w2_ref, c_ref, h_ref):
    h_ref[...] = _mlp_core(x_ref, p0_ref, p1_ref, w1_ref, w2_ref, c_ref)


def _mlp_mid(h, partials, w1, w2, coefs, bond_next):
    return pl.pallas_call(
        _mlp_mid_body,
        grid=(NBLK,),
        in_specs=[
            pl.BlockSpec((ROWB, D), lambda i: (i, 0)),
            pl.BlockSpec((1, ROWB, D), lambda i: (0, i, 0)),
            pl.BlockSpec((1, ROWB, D), lambda i: (1, i, 0)),
            pl.BlockSpec((D, D), lambda i: (0, 0)),
            pl.BlockSpec((D, D), lambda i: (0, 0)),
            pl.BlockSpec((8, D), lambda i: (0, 0)),
            pl.BlockSpec((8, D), lambda i: (0, 0)),
        ],
        out_shape=[
            jax.ShapeDtypeStruct((NPAD, D), jnp.float32),
            jax.ShapeDtypeStruct((4, NPAD, D), jnp.float32),
        ],
        out_specs=[
            pl.BlockSpec((ROWB, D), lambda i: (i, 0)),
            pl.BlockSpec((4, ROWB, D), lambda i: (0, i, 0)),
        ],
    )(h, partials, partials, w1, w2, coefs, bond_next)


def _mlp_last(h, partials, w1, w2, coefs):
    return pl.pallas_call(
        _mlp_last_body,
        grid=(NBLK,),
        in_specs=[
            pl.BlockSpec((ROWB, D), lambda i: (i, 0)),
            pl.BlockSpec((1, ROWB, D), lambda i: (0, i, 0)),
            pl.BlockSpec((1, ROWB, D), lambda i: (1, i, 0)),
            pl.BlockSpec((D, D), lambda i: (0, 0)),
            pl.BlockSpec((D, D), lambda i: (0, 0)),
            pl.BlockSpec((8, D), lambda i: (0, 0)),
        ],
        out_shape=jax.ShapeDtypeStruct((NPAD, D), jnp.float32),
        out_specs=pl.BlockSpec((ROWB, D), lambda i: (i, 0)),
    )(h, partials, partials, w1, w2, coefs)


ZB = 64


def _sc_scatter(mtab, idx_r, dst_r):
    mesh = plsc.VectorSubcoreMesh(core_axis_name="c", subcore_axis_name="s")

    @functools.partial(
        pl.kernel,
        out_type=jax.ShapeDtypeStruct((NCORES, NPAD, D), jnp.float32),
        mesh=mesh,
        scratch_types=[
            pltpu.VMEM((G, CHUNK), jnp.int32),
            pltpu.VMEM((G, CHUNK), jnp.int32),
            [pltpu.VMEM((CHUNK, D), jnp.float32) for _ in range(NBUF)],
            pltpu.VMEM((ZB, D), jnp.float32),
            pltpu.VMEM_SHARED((NPAD, D), jnp.float32),
            [pltpu.SemaphoreType.DMA for _ in range(NBUF)],
        ],
    )
    def body(mtab_hbm, idx_hbm, dst_hbm, out_hbm,
             idx_v, dst_v, bufs, zb, agg_sh, sems):
        cid = lax.axis_index("c")
        sid = lax.axis_index("s")
        wid = cid * NSUB + sid
        zv = jnp.zeros((16,), jnp.float32)
        for r in range(ZB):
            for j in range(D // 16):
                zb[r, pl.ds(j * 16, 16)] = zv
        for t in range(RPT // ZB):
            pltpu.sync_copy(zb, agg_sh.at[pl.ds(sid * RPT + t * ZB, ZB)])
        plsc.subcore_barrier()

        def group(g, carry):
            pltpu.sync_copy(idx_hbm.at[wid, g], idx_v)
            pltpu.sync_copy(dst_hbm.at[wid, g], dst_v)
            for c in range(NBUF - 1):
                pltpu.async_copy(mtab_hbm.at[idx_v.at[c]], bufs[c], sems[c])

            def step(t, inner):
                for b in range(NBUF):
                    c = NBUF * t + b
                    nc = c + NBUF - 1

                    nb = (b + NBUF - 1) % NBUF

                    @pl.when(nc < G)
                    def _():
                        pltpu.async_copy(mtab_hbm.at[idx_v.at[nc]],
                                         bufs[nb], sems[nb])

                    pltpu.make_async_copy(mtab_hbm.at[idx_v.at[c]],
                                          bufs[b], sems[b]).wait()
                    pltpu.sync_copy(bufs[b], agg_sh.at[dst_v.at[c]], add=True)
                return inner

            lax.fori_loop(0, G // NBUF, step, 0)
            return carry

        lax.fori_loop(0, NG, group, 0)
        plsc.subcore_barrier()
        pltpu.sync_copy(agg_sh.at[pl.ds(sid * RPT, RPT)],
                        out_hbm.at[cid, pl.ds(sid * RPT, RPT)])

    return body(mtab, idx_r, dst_r)


def kernel(x, params, edge_index, edge_attr):
    src = edge_index[0].astype(jnp.int32)
    dst = edge_index[1].astype(jnp.int32)
    attr = edge_attr.astype(jnp.int32)

    pad = EPAD - E
    pad_iota = jnp.arange(pad, dtype=jnp.int32)
    src_r = jnp.concatenate([src, pad_iota % N]).reshape(NW, NG, G, CHUNK)
    attr_r = jnp.pad(attr, (0, pad)).reshape(NW, NG, G, CHUNK)
    pad_dst = DUMMY + (pad_iota % (NPAD - N))
    dst_r = jnp.concatenate([dst, pad_dst]).reshape(NW, NG, G, CHUNK)

    idx_r = _make_idx(src_r, attr_r)

    h = jnp.pad(x, ((0, NPAD - N), (0, 0)))
    bn_s = jnp.float32(1.0 / jnp.sqrt(1.0 + BN_EPS))

    mtab = None
    for l in range(NUM_LAYERS):
        coefs = jnp.stack([
            params[f"b1_{l}"],
            params[f"b2_{l}"],
            params[f"mg_{l}"] * bn_s,
            params[f"mb_{l}"],
            params[f"og_{l}"] * bn_s,
            params[f"ob_{l}"],
            jnp.full((D,), 1.0, jnp.float32) * (1.0 + params[f"eps_{l}"][0]),
            jnp.zeros((D,), jnp.float32),
        ])
        if l == 0:
            bond0 = jnp.pad(params["bond_0"], ((0, 4), (0, 0)))
            mtab = _make_mtab0(h, bond0)
        partials = _sc_scatter(mtab.reshape(4 * NPAD, D), idx_r, dst_r)
        if l == NUM_LAYERS - 1:
            h = _mlp_last(h, partials, params[f"W1_{l}"], params[f"W2_{l}"], coefs)
        else:
            bond_next = jnp.pad(params[f"bond_{l + 1}"], ((0, 4), (0, 0)))
            h, mtab = _mlp_mid(h, partials, params[f"W1_{l}"], params[f"W2_{l}"],
                               coefs, bond_next)
    return h[:N]


# speedup vs baseline: 4.7423x; 1.0685x over previous
import functools

import jax
import jax.numpy as jnp
from jax import lax
from jax.experimental import pallas as pl
from jax.experimental.pallas import tpu as pltpu
from jax.experimental.pallas import tpu_sc as plsc

N = 10000
E = 320000
D = 128
NUM_LAYERS = 2
BN_EPS = 1e-5

NPAD = 10240
ROWB = 1024
NBLK = NPAD // ROWB
NCORES = 2
NSUB = 16
NW = NCORES * NSUB
CHUNK = 64
NBUF = 4
G = 32
NG = 5
CPT = NG * G
EPT = CPT * CHUNK
EPAD = NW * EPT
RPT = NPAD // NSUB
DUMMY = N


def _idx_body(src_ref, attr_ref, idx_ref):
    idx_ref[...] = attr_ref[...] * NPAD + src_ref[...]


def _make_idx(src_r, attr_r):
    return pl.pallas_call(
        _idx_body,
        out_shape=jax.ShapeDtypeStruct((NW, NG, G, CHUNK), jnp.int32),
    )(src_r, attr_r)


def _mtab0_body(x_ref, bond_ref, out_ref):
    h = x_ref[...]
    for a in range(4):
        out_ref[a] = jnp.maximum(h + bond_ref[a:a + 1, :], 0.0)


def _make_mtab0(x_pad, bond):
    return pl.pallas_call(
        _mtab0_body,
        grid=(NBLK,),
        in_specs=[
            pl.BlockSpec((ROWB, D), lambda i: (i, 0)),
            pl.BlockSpec((8, D), lambda i: (0, 0)),
        ],
        out_shape=jax.ShapeDtypeStruct((4, NPAD, D), jnp.float32),
        out_specs=pl.BlockSpec((4, ROWB, D), lambda i: (0, i, 0)),
    )(x_pad, bond)


def _mlp_core(x_ref, p0_ref, p1_ref, w1_ref, w2_ref, c_ref):
    z = x_ref[...] * c_ref[6:7, :] + p0_ref[0] + p1_ref[0]
    z = jnp.dot(z, w1_ref[...], preferred_element_type=jnp.float32) + c_ref[0:1, :]
    z = jnp.maximum(z * c_ref[2:3, :] + c_ref[3:4, :], 0.0)
    z = jnp.dot(z, w2_ref[...], preferred_element_type=jnp.float32) + c_ref[1:2, :]
    return z * c_ref[4:5, :] + c_ref[5:6, :]


def _mlp_mid_body(x_ref, p0_ref, p1_ref, w1_ref, w2_ref, c_ref, bond_ref,
                  h_ref, mt_ref):
    z = jnp.maximum(_mlp_core(x_ref, p0_ref, p1_ref, w1_ref, w2_ref, c_ref), 0.0)
    h_ref[...] = z
    for a in range(4):
        mt_ref[a] = jnp.maximum(z + bond_ref[a:a + 1, :], 0.0)


def _mlp_last_body(x_ref, p0_ref, p1_ref, w1_ref, w2_ref, c_ref, h_ref):
    h_ref[...] = _mlp_core(x_ref, p0_ref, p1_ref, w1_ref, w2_ref, c_ref)


def _mlp_mid(h, partials, w1, w2, coefs, bond_next):
    return pl.pallas_call(
        _mlp_mid_body,
        grid=(NBLK,),
        in_specs=[
            pl.BlockSpec((ROWB, D), lambda i: (i, 0)),
            pl.BlockSpec((1, ROWB, D), lambda i: (0, i, 0)),
            pl.BlockSpec((1, ROWB, D), lambda i: (1, i, 0)),
            pl.BlockSpec((D, D), lambda i: (0, 0)),
            pl.BlockSpec((D, D), lambda i: (0, 0)),
            pl.BlockSpec((8, D), lambda i: (0, 0)),
            pl.BlockSpec((8, D), lambda i: (0, 0)),
        ],
        out_shape=[
            jax.ShapeDtypeStruct((NPAD, D), jnp.float32),
            jax.ShapeDtypeStruct((4, NPAD, D), jnp.float32),
        ],
        out_specs=[
            pl.BlockSpec((ROWB, D), lambda i: (i, 0)),
            pl.BlockSpec((4, ROWB, D), lambda i: (0, i, 0)),
        ],
    )(h, partials, partials, w1, w2, coefs, bond_next)


def _mlp_last(h, partials, w1, w2, coefs):
    return pl.pallas_call(
        _mlp_last_body,
        grid=(NBLK,),
        in_specs=[
            pl.BlockSpec((ROWB, D), lambda i: (i, 0)),
            pl.BlockSpec((1, ROWB, D), lambda i: (0, i, 0)),
            pl.BlockSpec((1, ROWB, D), lambda i: (1, i, 0)),
            pl.BlockSpec((D, D), lambda i: (0, 0)),
            pl.BlockSpec((D, D), lambda i: (0, 0)),
            pl.BlockSpec((8, D), lambda i: (0, 0)),
        ],
        out_shape=jax.ShapeDtypeStruct((N, D), jnp.float32),
        out_specs=pl.BlockSpec((ROWB, D), lambda i: (i, 0)),
    )(h, partials, partials, w1, w2, coefs)


ZB = 64


def _sc_scatter(mtab, idx_r, dst_r):
    mesh = plsc.VectorSubcoreMesh(core_axis_name="c", subcore_axis_name="s")

    @functools.partial(
        pl.kernel,
        out_type=jax.ShapeDtypeStruct((NCORES, NPAD, D), jnp.float32),
        mesh=mesh,
        scratch_types=[
            [pltpu.VMEM((G, CHUNK), jnp.int32) for _ in range(2)],
            [pltpu.VMEM((G, CHUNK), jnp.int32) for _ in range(2)],
            [pltpu.VMEM((CHUNK, D), jnp.float32) for _ in range(NBUF)],
            pltpu.VMEM_SHARED((NPAD, D), jnp.float32),
            [pltpu.SemaphoreType.DMA for _ in range(NBUF)],
            [pltpu.SemaphoreType.DMA for _ in range(4)],
        ],
    )
    def body(mtab_hbm, idx_hbm, dst_hbm, out_hbm,
             idx_v, dst_v, bufs, agg_sh, sems, isems):
        cid = lax.axis_index("c")
        sid = lax.axis_index("s")
        wid = cid * NSUB + sid

        pltpu.async_copy(idx_hbm.at[wid, 0], idx_v[0], isems[0])
        pltpu.async_copy(dst_hbm.at[wid, 0], dst_v[0], isems[1])
        zb = bufs[NBUF - 1]
        zv = jnp.zeros((16,), jnp.float32)
        for r in range(CHUNK):
            for j in range(D // 16):
                zb[r, pl.ds(j * 16, 16)] = zv
        pltpu.make_async_copy(idx_hbm.at[wid, 0], idx_v[0], isems[0]).wait()
        for c in range(NBUF - 1):
            pltpu.async_copy(mtab_hbm.at[idx_v[0].at[c]], bufs[c], sems[c])
        for t in range(RPT // CHUNK):
            pltpu.sync_copy(zb, agg_sh.at[pl.ds(sid * RPT + t * CHUNK, CHUNK)])
        pltpu.make_async_copy(dst_hbm.at[wid, 0], dst_v[0], isems[1]).wait()
        plsc.subcore_barrier()

        for g in range(NG):
            par = g % 2
            onp = 1 - par
            nxt = g + 1
            if nxt < NG:
                pltpu.async_copy(idx_hbm.at[wid, nxt], idx_v[onp],
                                 isems[2 * onp])
                pltpu.async_copy(dst_hbm.at[wid, nxt], dst_v[onp],
                                 isems[2 * onp + 1])

            def step(t, inner, par=par):
                for b in range(NBUF):
                    c = NBUF * t + b
                    nc = c + NBUF - 1
                    nb = (b + NBUF - 1) % NBUF

                    @pl.when(nc < G)
                    def _():
                        pltpu.async_copy(mtab_hbm.at[idx_v[par].at[nc]],
                                         bufs[nb], sems[nb])

                    pltpu.make_async_copy(mtab_hbm.at[idx_v[par].at[c]],
                                          bufs[b], sems[b]).wait()
                    pltpu.sync_copy(bufs[b], agg_sh.at[dst_v[par].at[c]],
                                    add=True)
                return inner

            lax.fori_loop(0, G // NBUF, step, 0)

            if nxt < NG:
                pltpu.make_async_copy(idx_hbm.at[wid, nxt], idx_v[onp],
                                      isems[2 * onp]).wait()
                for c in range(NBUF - 1):
                    pltpu.async_copy(mtab_hbm.at[idx_v[onp].at[c]],
                                     bufs[c], sems[c])
                pltpu.make_async_copy(dst_hbm.at[wid, nxt], dst_v[onp],
                                      isems[2 * onp + 1]).wait()

        plsc.subcore_barrier()
        pltpu.sync_copy(agg_sh.at[pl.ds(sid * RPT, RPT)],
                        out_hbm.at[cid, pl.ds(sid * RPT, RPT)])

    return body(mtab, idx_r, dst_r)


def kernel(x, params, edge_index, edge_attr):
    src = edge_index[0].astype(jnp.int32)
    dst = edge_index[1].astype(jnp.int32)
    attr = edge_attr.astype(jnp.int32)

    pad = EPAD - E
    pad_iota = jnp.arange(pad, dtype=jnp.int32)
    src_r = jnp.concatenate([src, pad_iota % N]).reshape(NW, NG, G, CHUNK)
    attr_r = jnp.pad(attr, (0, pad)).reshape(NW, NG, G, CHUNK)
    pad_dst = DUMMY + (pad_iota % (NPAD - N))
    dst_r = jnp.concatenate([dst, pad_dst]).reshape(NW, NG, G, CHUNK)

    idx_r = _make_idx(src_r, attr_r)

    h = jnp.pad(x, ((0, NPAD - N), (0, 0)))
    bn_s = jnp.float32(1.0 / jnp.sqrt(1.0 + BN_EPS))

    mtab = None
    for l in range(NUM_LAYERS):
        coefs = jnp.stack([
            params[f"b1_{l}"],
            params[f"b2_{l}"],
            params[f"mg_{l}"] * bn_s,
            params[f"mb_{l}"],
            params[f"og_{l}"] * bn_s,
            params[f"ob_{l}"],
            jnp.full((D,), 1.0, jnp.float32) * (1.0 + params[f"eps_{l}"][0]),
            jnp.zeros((D,), jnp.float32),
        ])
        if l == 0:
            bond0 = jnp.pad(params["bond_0"], ((0, 4), (0, 0)))
            mtab = _make_mtab0(h, bond0)
        partials = _sc_scatter(mtab.reshape(4 * NPAD, D), idx_r, dst_r)
        if l == NUM_LAYERS - 1:
            h = _mlp_last(h, partials, params[f"W1_{l}"], params[f"W2_{l}"], coefs)
        else:
            bond_next = jnp.pad(params[f"bond_{l + 1}"], ((0, 4), (0, 0)))
            h, mtab = _mlp_mid(h, partials, params[f"W1_{l}"], params[f"W2_{l}"],
                               coefs, bond_next)
    return h


# speedup vs baseline: 4.9188x; 1.0372x over previous
import functools

import jax
import jax.numpy as jnp
from jax import lax
from jax.experimental import pallas as pl
from jax.experimental.pallas import tpu as pltpu
from jax.experimental.pallas import tpu_sc as plsc

N = 10000
E = 320000
D = 128
NUM_LAYERS = 2
BN_EPS = 1e-5

NPAD = 10240
ROWB = 1024
NBLK = NPAD // ROWB
NCORES = 2
NSUB = 16
NW = NCORES * NSUB
CHUNK = 64
NBUF = 4
G = 32
NG = 5
CPT = NG * G
EPT = CPT * CHUNK
EPAD = NW * EPT
RPT = NPAD // NSUB
DUMMY = N


def _idx_body(src_ref, attr_ref, idx_ref):
    idx_ref[...] = attr_ref[...] * NPAD + src_ref[...]


def _make_idx(src_r, attr_r):
    return pl.pallas_call(
        _idx_body,
        out_shape=jax.ShapeDtypeStruct((NW, NG, G, CHUNK), jnp.int32),
    )(src_r, attr_r)


def _mtab0_body(x_ref, bond_ref, out_ref):
    h = x_ref[...]
    for a in range(4):
        out_ref[a] = jnp.maximum(h + bond_ref[a:a + 1, :], 0.0)


def _make_mtab0(x_pad, bond):
    return pl.pallas_call(
        _mtab0_body,
        grid=(NBLK,),
        in_specs=[
            pl.BlockSpec((ROWB, D), lambda i: (i, 0)),
            pl.BlockSpec((8, D), lambda i: (0, 0)),
        ],
        out_shape=jax.ShapeDtypeStruct((4, NPAD, D), jnp.float32),
        out_specs=pl.BlockSpec((4, ROWB, D), lambda i: (0, i, 0)),
    )(x_pad, bond)


def _mlp_core(x_ref, p0_ref, p1_ref, w1_ref, w2_ref, c_ref):
    z = x_ref[...] * c_ref[6:7, :] + p0_ref[0] + p1_ref[0]
    z = jnp.dot(z, w1_ref[...], preferred_element_type=jnp.float32) + c_ref[0:1, :]
    z = jnp.maximum(z * c_ref[2:3, :] + c_ref[3:4, :], 0.0)
    z = jnp.dot(z, w2_ref[...], preferred_element_type=jnp.float32) + c_ref[1:2, :]
    return z * c_ref[4:5, :] + c_ref[5:6, :]


def _mlp_mid_body(x_ref, p0_ref, p1_ref, w1_ref, w2_ref, c_ref, bond_ref,
                  h_ref, mt_ref):
    z = jnp.maximum(_mlp_core(x_ref, p0_ref, p1_ref, w1_ref, w2_ref, c_ref), 0.0)
    h_ref[...] = z
    for a in range(4):
        mt_ref[a] = jnp.maximum(z + bond_ref[a:a + 1, :], 0.0)


def _mlp_last_body(x_ref, p0_ref, p1_ref, w1_ref, w2_ref, c_ref, h_ref):
    h_ref[...] = _mlp_core(x_ref, p0_ref, p1_ref, w1_ref, w2_ref, c_ref)


def _mlp_mid(h, partials, w1, w2, coefs, bond_next):
    return pl.pallas_call(
        _mlp_mid_body,
        grid=(NBLK,),
        in_specs=[
            pl.BlockSpec((ROWB, D), lambda i: (i, 0)),
            pl.BlockSpec((1, ROWB, D), lambda i: (0, i, 0)),
            pl.BlockSpec((1, ROWB, D), lambda i: (1, i, 0)),
            pl.BlockSpec((D, D), lambda i: (0, 0)),
            pl.BlockSpec((D, D), lambda i: (0, 0)),
            pl.BlockSpec((8, D), lambda i: (0, 0)),
            pl.BlockSpec((8, D), lambda i: (0, 0)),
        ],
        out_shape=[
            jax.ShapeDtypeStruct((NPAD, D), jnp.float32),
            jax.ShapeDtypeStruct((4, NPAD, D), jnp.float32),
        ],
        out_specs=[
            pl.BlockSpec((ROWB, D), lambda i: (i, 0)),
            pl.BlockSpec((4, ROWB, D), lambda i: (0, i, 0)),
        ],
    )(h, partials, partials, w1, w2, coefs, bond_next)


def _mlp_last(h, partials, w1, w2, coefs):
    return pl.pallas_call(
        _mlp_last_body,
        grid=(NBLK,),
        in_specs=[
            pl.BlockSpec((ROWB, D), lambda i: (i, 0)),
            pl.BlockSpec((1, ROWB, D), lambda i: (0, i, 0)),
            pl.BlockSpec((1, ROWB, D), lambda i: (1, i, 0)),
            pl.BlockSpec((D, D), lambda i: (0, 0)),
            pl.BlockSpec((D, D), lambda i: (0, 0)),
            pl.BlockSpec((8, D), lambda i: (0, 0)),
        ],
        out_shape=jax.ShapeDtypeStruct((N, D), jnp.float32),
        out_specs=pl.BlockSpec((ROWB, D), lambda i: (i, 0)),
    )(h, partials, partials, w1, w2, coefs)


def _sc_scatter(mtab, idx_r, dst_r):
    mesh = plsc.VectorSubcoreMesh(core_axis_name="c", subcore_axis_name="s")

    @functools.partial(
        pl.kernel,
        out_type=jax.ShapeDtypeStruct((NCORES, NPAD, D), jnp.float32),
        mesh=mesh,
        scratch_types=[
            [pltpu.VMEM((G, CHUNK), jnp.int32) for _ in range(2)],
            [pltpu.VMEM((G, CHUNK), jnp.int32) for _ in range(2)],
            [pltpu.VMEM((CHUNK, D), jnp.float32) for _ in range(NBUF)],
            pltpu.VMEM_SHARED((NPAD, D), jnp.float32),
            [pltpu.SemaphoreType.DMA for _ in range(NBUF)],
            [pltpu.SemaphoreType.DMA for _ in range(NBUF)],
            [pltpu.SemaphoreType.DMA for _ in range(4)],
        ],
    )
    def body(mtab_hbm, idx_hbm, dst_hbm, out_hbm,
             idx_v, dst_v, bufs, agg_sh, sems, ssems, isems):
        cid = lax.axis_index("c")
        sid = lax.axis_index("s")
        wid = cid * NSUB + sid

        pltpu.async_copy(idx_hbm.at[wid, 0], idx_v[0], isems[0])
        pltpu.async_copy(dst_hbm.at[wid, 0], dst_v[0], isems[1])
        zb = bufs[NBUF - 1]
        zv = jnp.zeros((16,), jnp.float32)
        for r in range(CHUNK):
            for j in range(D // 16):
                zb[r, pl.ds(j * 16, 16)] = zv
        pltpu.make_async_copy(idx_hbm.at[wid, 0], idx_v[0], isems[0]).wait()
        for c in range(NBUF - 1):
            pltpu.async_copy(mtab_hbm.at[idx_v[0].at[c]], bufs[c], sems[c])
        for t in range(RPT // CHUNK):
            pltpu.sync_copy(zb, agg_sh.at[pl.ds(sid * RPT + t * CHUNK, CHUNK)])
        pltpu.make_async_copy(dst_hbm.at[wid, 0], dst_v[0], isems[1]).wait()
        plsc.subcore_barrier()

        for g in range(NG):
            par = g % 2
            onp = 1 - par
            nxt = g + 1
            if nxt < NG:
                pltpu.async_copy(idx_hbm.at[wid, nxt], idx_v[onp],
                                 isems[2 * onp])
                pltpu.async_copy(dst_hbm.at[wid, nxt], dst_v[onp],
                                 isems[2 * onp + 1])

            def step(t, inner, par=par, g=g):
                for b in range(NBUF):
                    c = NBUF * t + b
                    nc = c + NBUF - 1
                    nb = (b + NBUF - 1) % NBUF

                    def _start_next():
                        pltpu.make_async_copy(
                            bufs[nb], agg_sh.at[dst_v[par].at[0]],
                            ssems[nb]).wait()
                        pltpu.async_copy(mtab_hbm.at[idx_v[par].at[nc]],
                                         bufs[nb], sems[nb])

                    if g == 0 and b == 0:
                        @pl.when((nc < G) & (t > 0))
                        def _():
                            _start_next()

                        @pl.when((nc < G) & (t == 0))
                        def _():
                            pltpu.async_copy(mtab_hbm.at[idx_v[par].at[nc]],
                                             bufs[nb], sems[nb])
                    else:
                        @pl.when(nc < G)
                        def _():
                            _start_next()

                    pltpu.make_async_copy(mtab_hbm.at[idx_v[par].at[c]],
                                          bufs[b], sems[b]).wait()
                    pltpu.async_copy(bufs[b], agg_sh.at[dst_v[par].at[c]],
                                     ssems[b], add=True)
                return inner

            lax.fori_loop(0, G // NBUF, step, 0)

            if nxt < NG:
                pltpu.make_async_copy(idx_hbm.at[wid, nxt], idx_v[onp],
                                      isems[2 * onp]).wait()
                for c in range(NBUF - 1):
                    pltpu.make_async_copy(bufs[c], agg_sh.at[dst_v[par].at[0]],
                                          ssems[c]).wait()
                    pltpu.async_copy(mtab_hbm.at[idx_v[onp].at[c]],
                                     bufs[c], sems[c])
                pltpu.make_async_copy(dst_hbm.at[wid, nxt], dst_v[onp],
                                      isems[2 * onp + 1]).wait()
            else:
                for b in range(NBUF):
                    pltpu.make_async_copy(bufs[b], agg_sh.at[dst_v[par].at[0]],
                                          ssems[b]).wait()

        plsc.subcore_barrier()
        pltpu.sync_copy(agg_sh.at[pl.ds(sid * RPT, RPT)],
                        out_hbm.at[cid, pl.ds(sid * RPT, RPT)])

    return body(mtab, idx_r, dst_r)


def kernel(x, params, edge_index, edge_attr):
    src = edge_index[0].astype(jnp.int32)
    dst = edge_index[1].astype(jnp.int32)
    attr = edge_attr.astype(jnp.int32)

    pad = EPAD - E
    pad_iota = jnp.arange(pad, dtype=jnp.int32)
    src_r = jnp.concatenate([src, pad_iota % N]).reshape(NW, NG, G, CHUNK)
    attr_r = jnp.pad(attr, (0, pad)).reshape(NW, NG, G, CHUNK)
    pad_dst = DUMMY + (pad_iota % (NPAD - N))
    dst_r = jnp.concatenate([dst, pad_dst]).reshape(NW, NG, G, CHUNK)

    idx_r = _make_idx(src_r, attr_r)

    h = jnp.pad(x, ((0, NPAD - N), (0, 0)))
    bn_s = jnp.float32(1.0 / jnp.sqrt(1.0 + BN_EPS))

    mtab = None
    for l in range(NUM_LAYERS):
        coefs = jnp.stack([
            params[f"b1_{l}"],
            params[f"b2_{l}"],
            params[f"mg_{l}"] * bn_s,
            params[f"mb_{l}"],
            params[f"og_{l}"] * bn_s,
            params[f"ob_{l}"],
            jnp.full((D,), 1.0, jnp.float32) * (1.0 + params[f"eps_{l}"][0]),
            jnp.zeros((D,), jnp.float32),
        ])
        if l == 0:
            bond0 = jnp.pad(params["bond_0"], ((0, 4), (0, 0)))
            mtab = _make_mtab0(h, bond0)
        partials = _sc_scatter(mtab.reshape(4 * NPAD, D), idx_r, dst_r)
        if l == NUM_LAYERS - 1:
            h = _mlp_last(h, partials, params[f"W1_{l}"], params[f"W2_{l}"], coefs)
        else:
            bond_next = jnp.pad(params[f"bond_{l + 1}"], ((0, 4), (0, 0)))
            h, mtab = _mlp_mid(h, partials, params[f"W1_{l}"], params[f"W2_{l}"],
                               coefs, bond_next)
    return h
